# baseline (speedup 1.0000x reference)
import functools

import jax
import jax.numpy as jnp
from jax import lax
from jax.experimental import pallas as pl
from jax.experimental.pallas import tpu as pltpu
from jax.experimental.pallas import tpu_sc as plsc

N = 320000
C_IN = 128
UNITS = 64
NSEG = 10000
NW = 32
SEG_PER_W = 320
VPAD = NW * SEG_PER_W
MFROWS = VPAD

BN = 4000

def _mm_body(x_ref, w_ref, b_ref, o_ref):
    y = lax.dot_general(
        x_ref[...], w_ref[...], (((1,), (1,)), ((), ())),
        preferred_element_type=jnp.float32) + b_ref[...]
    o_ref[:, 0:UNITS] = y
    o_ref[:, UNITS:2 * UNITS] = y


def _matmul(inputs, W, b2d):
    return pl.pallas_call(
        _mm_body,
        grid=(N // BN,),
        in_specs=[
            pl.BlockSpec((BN, C_IN), lambda i: (i, 0)),
            pl.BlockSpec((UNITS, C_IN), lambda i: (0, 0)),
            pl.BlockSpec((1, UNITS), lambda i: (0, 0)),
        ],
        out_specs=pl.BlockSpec((BN, 2 * UNITS), lambda i: (i, 0)),
        out_shape=jax.ShapeDtypeStruct((N, 2 * UNITS), jnp.float32),
    )(inputs, W, b2d)


NRANGE = 8
SEG_PER_R = VPAD // NRANGE
ROWS_PER_C = N // NW
CH2 = 80
NSC = ROWS_PER_C // CH2
MB = 80

_mesh = plsc.VectorSubcoreMesh(core_axis_name="c", subcore_axis_name="s")


@functools.partial(
    pl.kernel,
    out_type=jax.ShapeDtypeStruct((MFROWS, 2 * UNITS), jnp.float32),
    mesh=_mesh,
    scratch_types=[
        pltpu.VMEM((CH2,), jnp.int32),
        pltpu.VMEM((CH2,), jnp.int32),
        pltpu.VMEM((CH2, 2 * UNITS), jnp.float32),
        pltpu.VMEM((CH2, 2 * UNITS), jnp.float32),
        pltpu.VMEM(((SEG_PER_R + 4) * UNITS,), jnp.float32),
        pltpu.VMEM((MB * UNITS,), jnp.float32),
        pltpu.VMEM((MB * UNITS,), jnp.float32),
        pltpu.VMEM_SHARED((16 * 160 * UNITS,), jnp.float32),
        pltpu.SemaphoreType.DMA,
        pltpu.SemaphoreType.DMA,
        pltpu.SemaphoreType.DMA,
        pltpu.SemaphoreType.DMA,
    ],
)
def _segmax_kernel(xx_hbm, idx_hbm, mf_hbm, ib0, ib1, xb0, xb1, acc, hb0, hb1,
                   shr, si0, si1, sx0, sx1):
    wid = lax.axis_index("s") * 2 + lax.axis_index("c")
    g = lax.rem(wid, NRANGE)
    kq = wid // NRANGE
    lo = g * SEG_PER_R
    hi = lo + SEG_PER_R

    def init_acc(s, _):
        for f in range(UNITS // 16):
            acc[pl.ds(s * UNITS + f * 16, 16)] = jnp.full((16,), -jnp.inf,
                                                          jnp.float32)
        return 0
    lax.fori_loop(0, SEG_PER_R, init_acc, 0)

    TOT = NRANGE * NSC
    ibs = (ib0, ib1)
    xbs = (xb0, xb1)
    sis = (si0, si1)
    sxs = (sx0, sx1)

    def _off(j):
        p = j // NSC
        sc = j - p * NSC
        chunk = lax.rem(wid + p, NW)
        return chunk * ROWS_PER_C + sc * CH2

    for b in range(2):
        r0 = _off(jnp.int32(b))
        pltpu.async_copy(idx_hbm.at[pl.ds(r0, CH2)], ibs[b], sis[b])
        pltpu.async_copy(xx_hbm.at[pl.ds(r0, CH2)], xbs[b], sxs[b])

    def ring_body(s2, _):
        for b in range(2):
            j = s2 * 2 + b
            ibuf, xbuf = ibs[b], xbs[b]
            pltpu.make_async_copy(idx_hbm.at[pl.ds(0, CH2)], ibuf, sis[b]).wait()
            pltpu.make_async_copy(xx_hbm.at[pl.ds(0, CH2)], xbuf, sxs[b]).wait()
            def grp_body(vi, _):
                u = ibuf[pl.ds(vi * 16, 16)] - lo
                for t in range(16):
                    uj = u[t]

                    @pl.when((uj | (SEG_PER_R - 1 - uj)) >= 0)
                    def _(uj=uj, t=t):
                        for f in range(UNITS // 16):
                            al = pl.ds(uj * UNITS + f * 16, 16)
                            sl = pl.ds(f * 16, 16)
                            acc[al] = jnp.maximum(acc[al],
                                                  xbuf[vi * 16 + t, sl])
                return 0
            lax.fori_loop(0, CH2 // 16, grp_body, 0)
            rn = _off(jnp.minimum(j + 2, TOT - 1))
            pltpu.async_copy(idx_hbm.at[pl.ds(rn, CH2)], ibuf, sis[b])
            pltpu.async_copy(xx_hbm.at[pl.ds(rn, CH2)], xbuf, sxs[b])
        return 0

    lax.fori_loop(0, TOT // 2, ring_body, 0)
    for b in range(2):
        pltpu.make_async_copy(idx_hbm.at[pl.ds(0, CH2)], ibs[b], sis[b]).wait()
        pltpu.make_async_copy(xx_hbm.at[pl.ds(0, CH2)], xbs[b], sxs[b]).wait()

    sid = lax.axis_index("s")
    c = lax.axis_index("c")
    q = (g - c) // 2
    for k in range(8):
        pltpu.sync_copy(acc.at[pl.ds(k * 160 * UNITS, 160 * UNITS)],
                        shr.at[pl.ds(sid * 160 * UNITS, 160 * UNITS)])
        plsc.subcore_barrier()

        @pl.when(kq == k // 2)
        def _():
            def sb_body(sb, _):
                pltpu.sync_copy(
                    shr.at[pl.ds((q * 160 + sb * MB) * UNITS, MB * UNITS)], hb0)
                for h in range(1, 4):
                    src0 = ((4 * h + q) * 160 + sb * MB) * UNITS
                    pltpu.sync_copy(shr.at[pl.ds(src0, MB * UNITS)], hb1)

                    def mx_rows(r, _):
                        for f in range(UNITS // 16):
                            hl = pl.ds(r * UNITS + f * 16, 16)
                            hb0[hl] = jnp.maximum(hb0[hl], hb1[hl])
                        return 0
                    lax.fori_loop(0, MB, mx_rows, 0)

                def st_rows(r, _):
                    for f in range(UNITS // 16):
                        xb0[r, pl.ds(f * 16, 16)] = hb0[pl.ds(r * UNITS + f * 16, 16)]
                    return 0
                lax.fori_loop(0, MB, st_rows, 0)

                pltpu.sync_copy(
                    xb0, mf_hbm.at[pl.ds(lo + k * 160 + sb * MB, MB)])
                return 0
            lax.fori_loop(0, 160 // MB, sb_body, 0)

        plsc.subcore_barrier()


RB = 80
ROWS_PER_W = N // NW
NRB = ROWS_PER_W // RB


@functools.partial(
    pl.kernel,
    out_type=jax.ShapeDtypeStruct((N, 2 * UNITS), jnp.float32),
    mesh=_mesh,
    scratch_types=[
        pltpu.VMEM((RB,), jnp.int32),
        pltpu.VMEM((RB, 2 * UNITS), jnp.float32),
        pltpu.VMEM((RB, 2 * UNITS), jnp.float32),
        pltpu.VMEM((RB, 2 * UNITS), jnp.float32),
        pltpu.SemaphoreType.DMA,
    ],
)
def _gather_kernel(xx_hbm, idx_hbm, mf_hbm, out_hbm, idxv, xbuf, gbuf, obuf, sem):
    wid = lax.axis_index("s") * 2 + lax.axis_index("c")
    base = wid * ROWS_PER_W

    def chunk_body(c, _):
        r0 = base + c * RB
        pltpu.sync_copy(idx_hbm.at[pl.ds(r0, RB)], idxv)
        cp = pltpu.async_copy(mf_hbm.at[idxv], gbuf, sem)
        pltpu.sync_copy(xx_hbm.at[pl.ds(r0, RB)], xbuf)
        cp.wait()

        def asm_body(j, _):
            for f in range(UNITS // 16):
                sl = pl.ds(f * 16, 16)
                obuf[j, sl] = xbuf[j, sl]
                obuf[j, pl.ds(UNITS + f * 16, 16)] = gbuf[j, sl]
            return 0
        lax.fori_loop(0, RB, asm_body, 0)

        pltpu.sync_copy(obuf, out_hbm.at[pl.ds(r0, RB)])
        return 0

    lax.fori_loop(0, NRB, chunk_body, 0)


def kernel(inputs, idx_used, sizes, W, b):
    xx = _matmul(inputs, W, b.reshape(1, UNITS))
    mf = _segmax_kernel(xx, idx_used)
    out = _gather_kernel(xx, idx_used, mf)
    return out


# speedup vs baseline: 1.0301x; 1.0301x over previous
import functools

import jax
import jax.numpy as jnp
from jax import lax
from jax.experimental import pallas as pl
from jax.experimental.pallas import tpu as pltpu
from jax.experimental.pallas import tpu_sc as plsc

N = 320000
C_IN = 128
UNITS = 64
NSEG = 10000
NW = 32
SEG_PER_W = 320
VPAD = NW * SEG_PER_W
MFROWS = VPAD

BN = 4000

def _mm_body(x_ref, w_ref, b_ref, o_ref):
    y = lax.dot_general(
        x_ref[...], w_ref[...], (((1,), (1,)), ((), ())),
        preferred_element_type=jnp.float32) + b_ref[...]
    o_ref[:, 0:UNITS] = y
    o_ref[:, UNITS:2 * UNITS] = y


def _matmul(inputs, W, b2d):
    return pl.pallas_call(
        _mm_body,
        grid=(N // BN,),
        in_specs=[
            pl.BlockSpec((BN, C_IN), lambda i: (i, 0)),
            pl.BlockSpec((UNITS, C_IN), lambda i: (0, 0)),
            pl.BlockSpec((1, UNITS), lambda i: (0, 0)),
        ],
        out_specs=pl.BlockSpec((BN, 2 * UNITS), lambda i: (i, 0)),
        out_shape=jax.ShapeDtypeStruct((N, 2 * UNITS), jnp.float32),
    )(inputs, W, b2d)


NRANGE = 8
SEG_PER_R = VPAD // NRANGE
ROWS_PER_C = N // NW
CH2 = 80
NSC = ROWS_PER_C // CH2
MB = 80

_mesh = plsc.VectorSubcoreMesh(core_axis_name="c", subcore_axis_name="s")


@functools.partial(
    pl.kernel,
    out_type=jax.ShapeDtypeStruct((MFROWS, 2 * UNITS), jnp.float32),
    mesh=_mesh,
    scratch_types=[
        pltpu.VMEM((CH2,), jnp.int32),
        pltpu.VMEM((CH2,), jnp.int32),
        pltpu.VMEM((CH2, 2 * UNITS), jnp.float32),
        pltpu.VMEM((CH2, 2 * UNITS), jnp.float32),
        pltpu.VMEM(((SEG_PER_R + 4) * UNITS,), jnp.float32),
        pltpu.VMEM((MB * UNITS,), jnp.float32),
        pltpu.VMEM((MB * UNITS,), jnp.float32),
        pltpu.VMEM_SHARED((16 * 160 * UNITS,), jnp.float32),
        pltpu.SemaphoreType.DMA,
        pltpu.SemaphoreType.DMA,
        pltpu.SemaphoreType.DMA,
        pltpu.SemaphoreType.DMA,
    ],
)
def _segmax_kernel(xx_hbm, idx_hbm, mf_hbm, ib0, ib1, xb0, xb1, acc, hb0, hb1,
                   shr, si0, si1, sx0, sx1):
    wid = lax.axis_index("s") * 2 + lax.axis_index("c")
    g = lax.rem(wid, NRANGE)
    kq = wid // NRANGE
    lo = g * SEG_PER_R
    hi = lo + SEG_PER_R

    def init_acc(s, _):
        for f in range(UNITS // 16):
            acc[pl.ds(s * UNITS + f * 16, 16)] = jnp.full((16,), -jnp.inf,
                                                          jnp.float32)
        return 0
    lax.fori_loop(0, SEG_PER_R, init_acc, 0)

    TOT = NRANGE * NSC
    ibs = (ib0, ib1)
    xbs = (xb0, xb1)
    sis = (si0, si1)
    sxs = (sx0, sx1)

    def _off(j):
        p = j // NSC
        sc = j - p * NSC
        chunk = lax.rem(wid + p, NW)
        return chunk * ROWS_PER_C + sc * CH2

    for b in range(2):
        r0 = _off(jnp.int32(b))
        pltpu.async_copy(idx_hbm.at[pl.ds(r0, CH2)], ibs[b], sis[b])
        pltpu.async_copy(xx_hbm.at[pl.ds(r0, CH2)], xbs[b], sxs[b])

    def ring_body(s2, _):
        for b in range(2):
            j = s2 * 2 + b
            ibuf, xbuf = ibs[b], xbs[b]
            pltpu.make_async_copy(idx_hbm.at[pl.ds(0, CH2)], ibuf, sis[b]).wait()
            pltpu.make_async_copy(xx_hbm.at[pl.ds(0, CH2)], xbuf, sxs[b]).wait()
            def grp_body(vi, _):
                u = ibuf[pl.ds(vi * 16, 16)] - lo
                for t in range(16):
                    uj = u[t]
                    wj = uj | (SEG_PER_R - 1 - uj)
                    s = jnp.where(wj >= 0, uj, SEG_PER_R + (t & 3))
                    for f in range(UNITS // 16):
                        al = pl.ds(s * UNITS + f * 16, 16)
                        sl = pl.ds(f * 16, 16)
                        acc[al] = jnp.maximum(acc[al],
                                              xbuf[vi * 16 + t, sl])
                return 0
            lax.fori_loop(0, CH2 // 16, grp_body, 0)
            rn = _off(jnp.minimum(j + 2, TOT - 1))
            pltpu.async_copy(idx_hbm.at[pl.ds(rn, CH2)], ibuf, sis[b])
            pltpu.async_copy(xx_hbm.at[pl.ds(rn, CH2)], xbuf, sxs[b])
        return 0

    lax.fori_loop(0, TOT // 2, ring_body, 0)
    for b in range(2):
        pltpu.make_async_copy(idx_hbm.at[pl.ds(0, CH2)], ibs[b], sis[b]).wait()
        pltpu.make_async_copy(xx_hbm.at[pl.ds(0, CH2)], xbs[b], sxs[b]).wait()

    sid = lax.axis_index("s")
    c = lax.axis_index("c")
    q = (g - c) // 2
    for k in range(8):
        pltpu.sync_copy(acc.at[pl.ds(k * 160 * UNITS, 160 * UNITS)],
                        shr.at[pl.ds(sid * 160 * UNITS, 160 * UNITS)])
        plsc.subcore_barrier()

        @pl.when(kq == k // 2)
        def _():
            def sb_body(sb, _):
                pltpu.sync_copy(
                    shr.at[pl.ds((q * 160 + sb * MB) * UNITS, MB * UNITS)], hb0)
                for h in range(1, 4):
                    src0 = ((4 * h + q) * 160 + sb * MB) * UNITS
                    pltpu.sync_copy(shr.at[pl.ds(src0, MB * UNITS)], hb1)

                    def mx_rows(r, _):
                        for f in range(UNITS // 16):
                            hl = pl.ds(r * UNITS + f * 16, 16)
                            hb0[hl] = jnp.maximum(hb0[hl], hb1[hl])
                        return 0
                    lax.fori_loop(0, MB, mx_rows, 0)

                def st_rows(r, _):
                    for f in range(UNITS // 16):
                        xb0[r, pl.ds(f * 16, 16)] = hb0[pl.ds(r * UNITS + f * 16, 16)]
                    return 0
                lax.fori_loop(0, MB, st_rows, 0)

                pltpu.sync_copy(
                    xb0, mf_hbm.at[pl.ds(lo + k * 160 + sb * MB, MB)])
                return 0
            lax.fori_loop(0, 160 // MB, sb_body, 0)

        plsc.subcore_barrier()


RB = 80
ROWS_PER_W = N // NW
NRB = ROWS_PER_W // RB


@functools.partial(
    pl.kernel,
    out_type=jax.ShapeDtypeStruct((N, 2 * UNITS), jnp.float32),
    mesh=_mesh,
    scratch_types=[
        pltpu.VMEM((RB,), jnp.int32),
        pltpu.VMEM((RB,), jnp.int32),
        pltpu.VMEM((RB, 2 * UNITS), jnp.float32),
        pltpu.VMEM((RB, 2 * UNITS), jnp.float32),
        pltpu.VMEM((RB, 2 * UNITS), jnp.float32),
        pltpu.VMEM((RB, 2 * UNITS), jnp.float32),
        pltpu.VMEM((RB, 2 * UNITS), jnp.float32),
        pltpu.SemaphoreType.DMA,
        pltpu.SemaphoreType.DMA,
        pltpu.SemaphoreType.DMA,
        pltpu.SemaphoreType.DMA,
        pltpu.SemaphoreType.DMA,
        pltpu.SemaphoreType.DMA,
    ],
)
def _gather_kernel(xx_hbm, idx_hbm, mf_hbm, out_hbm, ib0, ib1, xb0, xb1,
                   gb0, gb1, obuf, si0, si1, sx0, sx1, sg0, sg1):
    wid = lax.axis_index("s") * 2 + lax.axis_index("c")
    base = wid * ROWS_PER_W
    ibs, xbs, gbs = (ib0, ib1), (xb0, xb1), (gb0, gb1)
    sis, sxs, sgs = (si0, si1), (sx0, sx1), (sg0, sg1)

    pltpu.async_copy(idx_hbm.at[pl.ds(base, RB)], ib0, si0)
    pltpu.async_copy(idx_hbm.at[pl.ds(base + RB, RB)], ib1, si1)
    pltpu.make_async_copy(idx_hbm.at[pl.ds(0, RB)], ib0, si0).wait()
    pltpu.async_copy(mf_hbm.at[ib0], gb0, sg0)
    pltpu.async_copy(xx_hbm.at[pl.ds(base, RB)], xb0, sx0)

    def _assemble(b, r0):
        xbuf, gbuf = xbs[b], gbs[b]

        def asm_body(j, _):
            for f in range(UNITS // 16):
                sl = pl.ds(f * 16, 16)
                obuf[j, sl] = xbuf[j, sl]
                obuf[j, pl.ds(UNITS + f * 16, 16)] = gbuf[j, sl]
            return 0
        lax.fori_loop(0, RB, asm_body, 0)
        pltpu.sync_copy(obuf, out_hbm.at[pl.ds(r0, RB)])

    def ring_body(s2, _):
        for b in range(2):
            c = s2 * 2 + b
            pltpu.make_async_copy(xx_hbm.at[pl.ds(0, RB)], xbs[b], sxs[b]).wait()
            pltpu.make_async_copy(mf_hbm.at[ibs[b]], gbs[b], sgs[b]).wait()
            _assemble(b, base + c * RB)
            cn = jnp.minimum(c + 2, NRB - 1)
            pltpu.async_copy(idx_hbm.at[pl.ds(base + cn * RB, RB)],
                             ibs[b], sis[b])
            pltpu.make_async_copy(idx_hbm.at[pl.ds(0, RB)],
                                  ibs[1 - b], sis[1 - b]).wait()
            pltpu.async_copy(mf_hbm.at[ibs[1 - b]], gbs[1 - b], sgs[1 - b])
            pltpu.async_copy(
                xx_hbm.at[pl.ds(base + jnp.minimum(c + 1, NRB - 1) * RB, RB)],
                xbs[1 - b], sxs[1 - b])
        return 0

    lax.fori_loop(0, (NRB - 1) // 2, ring_body, 0)
    pltpu.make_async_copy(xx_hbm.at[pl.ds(0, RB)], xb0, sx0).wait()
    pltpu.make_async_copy(mf_hbm.at[ib0], gb0, sg0).wait()
    _assemble(0, base + (NRB - 1) * RB)
    pltpu.make_async_copy(idx_hbm.at[pl.ds(0, RB)], ib1, si1).wait()


def kernel(inputs, idx_used, sizes, W, b):
    xx = _matmul(inputs, W, b.reshape(1, UNITS))
    mf = _segmax_kernel(xx, idx_used)
    out = _gather_kernel(xx, idx_used, mf)
    return out
